# baseline (speedup 1.0000x reference)
import functools

import jax
import jax.numpy as jnp
from jax import lax
from jax.experimental import pallas as pl
from jax.experimental.pallas import tpu as pltpu
from jax.experimental.pallas import tpu_sc as plsc

N_NODES = 10000
N_PAD = 10240
D_IN = 128
D_HID = 16
D_OUT = 3
N_EDGES = 320000
NC = 2
NS = 16
NW = NC * NS
CHUNK = 128
NCHUNK = 80
EPT = CHUNK * NCHUNK
E_PAD = NW * EPT
EPT2 = E_PAD // NS
RPS = N_PAD // NS
VR = N_PAD // 8
KB = 4
NBLK = NCHUNK // KB


def _sc_mesh():
  return plsc.VectorSubcoreMesh(core_axis_name="c", subcore_axis_name="s")


_SC_PARAMS = pltpu.CompilerParams(needs_layout_passes=False,
                                  use_tc_tiling_on_sc=False)


def _rsqrt16(d):
  bits = plsc.bitcast(d, jnp.int32)
  y = plsc.bitcast(jnp.int32(0x5F3759DF) - (bits >> 1), jnp.float32)
  for _ in range(3):
    y = y * (1.5 - 0.5 * d * y * y)
  return y


def _sc_deg(dst1d):
  @functools.partial(
      pl.kernel,
      out_type=jax.ShapeDtypeStruct((NC, N_PAD), jnp.float32),
      mesh=_sc_mesh(),
      compiler_params=_SC_PARAMS,
      scratch_types=[
          pltpu.VMEM((EPT2,), jnp.int32),
          pltpu.VMEM((N_PAD,), jnp.float32),
          pltpu.VMEM((RPS,), jnp.float32),
          pltpu.VMEM((RPS,), jnp.float32),
          pltpu.VMEM_SHARED((NS, N_PAD), jnp.float32),
      ],
  )
  def k(dst_hbm, dis_hbm, dst_v, hist_v, tmp_v, acc_v, hist_sh):
    c = lax.axis_index("c")
    s = lax.axis_index("s")
    zero16 = jnp.zeros((16,), jnp.float32)

    @pl.loop(0, N_PAD, step=16)
    def _(i):
      hist_v[pl.ds(i, 16)] = zero16

    pltpu.sync_copy(dst_hbm.at[pl.ds(s * EPT2, EPT2)], dst_v)

    @pl.loop(0, EPT2, step=64)
    def _(t):
      for u in (0, 16, 32, 48):
        idx = dst_v[pl.ds(t + u, 16)]
        cnt, last = plsc.scan_count(idx)
        plsc.addupdate_scatter(hist_v, [idx], cnt.astype(jnp.float32),
                               mask=last)

    pltpu.sync_copy(hist_v, hist_sh.at[s])
    plsc.subcore_barrier()

    @pl.loop(0, RPS, step=16)
    def _(i):
      acc_v[pl.ds(i, 16)] = zero16

    for j in range(NS):
      pltpu.sync_copy(hist_sh.at[j].at[pl.ds(s * RPS, RPS)], tmp_v)

      @pl.loop(0, RPS, step=16)
      def _(i):
        acc_v[pl.ds(i, 16)] += tmp_v[pl.ds(i, 16)]

    @pl.loop(0, RPS, step=16)
    def _(i):
      acc_v[pl.ds(i, 16)] = _rsqrt16(acc_v[pl.ds(i, 16)] + 1.0)

    pltpu.sync_copy(acc_v, dis_hbm.at[c].at[pl.ds(s * RPS, RPS)])

  return k(dst1d)


def _sc_agg(table, src2d, dst2d, dis2, zeros):
  @functools.partial(
      pl.kernel,
      out_type=jax.ShapeDtypeStruct((NC, N_PAD, D_HID), jnp.float32),
      mesh=_sc_mesh(),
      compiler_params=_SC_PARAMS,
      scratch_types=[
          pltpu.VMEM((NCHUNK, CHUNK), jnp.int32),
          pltpu.VMEM((NCHUNK, CHUNK), jnp.int32),
          pltpu.VMEM((KB, CHUNK, D_HID), jnp.float32),
          pltpu.VMEM((KB, CHUNK, D_HID), jnp.float32),
          pltpu.VMEM((RPS, D_HID), jnp.float32),
          pltpu.VMEM((RPS, D_HID), jnp.float32),
          pltpu.VMEM((RPS,), jnp.float32),
          pltpu.VMEM_SHARED((N_PAD, D_HID), jnp.float32),
          pltpu.VMEM_SHARED((N_PAD, D_HID), jnp.float32),
          pltpu.SemaphoreType.DMA,
          pltpu.SemaphoreType.DMA,
      ],
  )
  def k(tab_hbm, src_hbm, dst_hbm, dis_hbm, z_hbm, out_hbm, src_v, dst_v,
        buf_a, buf_b, tab_v, acc_v, dis_v, acc_sh, tab_sh, sem_a, sem_b):
    c = lax.axis_index("c")
    s = lax.axis_index("s")
    wid = c * NS + s
    rows = pl.ds(s * RPS, RPS)

    pltpu.sync_copy(z_hbm.at[rows], acc_sh.at[rows])
    pltpu.sync_copy(tab_hbm.at[rows], tab_v)
    pltpu.sync_copy(dis_hbm.at[c].at[rows], dis_v)
    pltpu.sync_copy(src_hbm.at[pl.ds(wid * NCHUNK, NCHUNK)], src_v)
    pltpu.sync_copy(dst_hbm.at[pl.ds(wid * NCHUNK, NCHUNK)], dst_v)

    @pl.loop(0, RPS)
    def _(r):
      dr = plsc.load_gather(dis_v, [jnp.full((16,), r, jnp.int32)])
      tab_v[r, :] = tab_v[r, :] * dr

    pltpu.sync_copy(tab_v, tab_sh.at[rows])
    plsc.subcore_barrier()

    def fire_g(b, bufs, sem):
      for t in range(KB):
        pltpu.async_copy(tab_sh.at[src_v.at[b * KB + t]], bufs.at[t], sem)

    def drain_g(bufs, sem):
      for t in range(KB):
        pltpu.make_async_copy(tab_hbm.at[src_v.at[0]], bufs.at[t], sem).wait()

    def scatter(b, bufs):
      for t in range(KB):
        pltpu.sync_copy(bufs.at[t], acc_sh.at[dst_v.at[b * KB + t]], add=True)

    fire_g(0, buf_a, sem_a)

    @pl.loop(0, NBLK - 2, step=2)
    def _(b):
      fire_g(b + 1, buf_b, sem_b)
      drain_g(buf_a, sem_a)
      scatter(b, buf_a)
      fire_g(b + 2, buf_a, sem_a)
      drain_g(buf_b, sem_b)
      scatter(b + 1, buf_b)

    fire_g(NBLK - 1, buf_b, sem_b)
    drain_g(buf_a, sem_a)
    scatter(NBLK - 2, buf_a)
    drain_g(buf_b, sem_b)
    scatter(NBLK - 1, buf_b)

    plsc.subcore_barrier()

    pltpu.sync_copy(acc_sh.at[rows], acc_v)
    f16 = jnp.where(jnp.broadcast_to(c, (16,)) == 0, 1.0, 0.0)

    @pl.loop(0, RPS)
    def _(r):
      dr = plsc.load_gather(dis_v, [jnp.full((16,), r, jnp.int32)])
      acc_v[r, :] = (acc_v[r, :] + f16 * tab_v[r, :]) * dr

    pltpu.sync_copy(acc_v, out_hbm.at[c].at[rows])

  return k(table, src2d, dst2d, dis2, zeros)


def _tc_mm(xv, w1b):
  def body(x_ref, w_ref, o_ref):
    o_ref[...] = jnp.dot(x_ref[...], w_ref[...],
                         preferred_element_type=jnp.float32)

  return pl.pallas_call(
      body,
      out_shape=jax.ShapeDtypeStruct((VR, 128), jnp.float32),
  )(xv, w1b)


def _tc_mid(q1, b1t):
  def body(q_ref, b_ref, o_ref):
    o_ref[...] = jnp.maximum(q_ref[0] + q_ref[1] + b_ref[...], 0.0)

  return pl.pallas_call(
      body,
      out_shape=jax.ShapeDtypeStruct((VR, 128), jnp.float32),
  )(q1, b1t)


def _tc_final(q2, w2b, b2t):
  def body(q_ref, w_ref, b_ref, o_ref):
    ah = q_ref[0] + q_ref[1]
    o_ref[...] = jnp.dot(ah, w_ref[...],
                         preferred_element_type=jnp.float32) + b_ref[...]

  return pl.pallas_call(
      body,
      out_shape=jax.ShapeDtypeStruct((VR, 8 * D_OUT), jnp.float32),
  )(q2, w2b, b2t)


def _block_diag(w, n):
  return jnp.kron(jnp.eye(n, dtype=w.dtype), w)


@jax.jit
def kernel(x, edge_index, W1, b1, W2, b2):
  x_pad = jnp.pad(x, ((0, N_PAD - N_NODES), (0, 0)))
  xv = x_pad.reshape(VR, 8 * D_IN)
  edges = jnp.pad(edge_index, ((0, 0), (0, E_PAD - N_EDGES)),
                  constant_values=N_PAD - 1)
  src2 = edges[0].reshape(NW * NCHUNK, CHUNK)
  dst2 = edges[1].reshape(NW * NCHUNK, CHUNK)
  z16 = jnp.zeros((N_PAD, D_HID), jnp.float32)
  w1b = _block_diag(W1, 8)
  w2b = _block_diag(W2, 8)
  b1t = jnp.tile(b1, 8).reshape(1, 128)
  b2t = jnp.tile(b2, 8).reshape(1, 8 * D_OUT)

  xw1 = _tc_mm(xv, w1b)
  dis2 = _sc_deg(edges[1])
  q1 = _sc_agg(xw1.reshape(N_PAD, D_HID), src2, dst2, dis2, z16)
  h = _tc_mid(q1.reshape(NC, VR, 128), b1t)
  q2 = _sc_agg(h.reshape(N_PAD, D_HID), src2, dst2, dis2, z16)
  out = _tc_final(q2.reshape(NC, VR, 128), w2b, b2t)
  return out.reshape(N_PAD, D_OUT)[:N_NODES]


# speedup vs baseline: 1.0109x; 1.0109x over previous
import functools

import jax
import jax.numpy as jnp
from jax import lax
from jax.experimental import pallas as pl
from jax.experimental.pallas import tpu as pltpu
from jax.experimental.pallas import tpu_sc as plsc

N_NODES = 10000
N_PAD = 10240
D_IN = 128
D_HID = 16
D_OUT = 3
N_EDGES = 320000
NC = 2
NS = 16
NW = NC * NS
CHUNK = 128
NCHUNK = 80
EPT = CHUNK * NCHUNK
E_PAD = NW * EPT
EPT3 = N_EDGES // NW
RPS = N_PAD // NS
VR = N_PAD // 8
KB = 4
NBLK = NCHUNK // KB


def _sc_mesh():
  return plsc.VectorSubcoreMesh(core_axis_name="c", subcore_axis_name="s")


_SC_PARAMS = pltpu.CompilerParams(needs_layout_passes=False,
                                  use_tc_tiling_on_sc=False)


def _rsqrt16(d):
  bits = plsc.bitcast(d, jnp.int32)
  y = plsc.bitcast(jnp.int32(0x5F3759DF) - (bits >> 1), jnp.float32)
  for _ in range(3):
    y = y * (1.5 - 0.5 * d * y * y)
  return y


def _sc_deg(dst1d):
  @functools.partial(
      pl.kernel,
      out_type=jax.ShapeDtypeStruct((NC, N_PAD), jnp.float32),
      mesh=_sc_mesh(),
      compiler_params=_SC_PARAMS,
      scratch_types=[
          pltpu.VMEM((EPT3,), jnp.int32),
          pltpu.VMEM((N_PAD,), jnp.float32),
          pltpu.VMEM((RPS,), jnp.float32),
          pltpu.VMEM((RPS,), jnp.float32),
          pltpu.VMEM_SHARED((NS, N_PAD), jnp.float32),
      ],
  )
  def k(dst_hbm, hist_hbm, dst_v, hist_v, tmp_v, acc_v, hist_sh):
    c = lax.axis_index("c")
    s = lax.axis_index("s")
    wid = c * NS + s
    zero16 = jnp.zeros((16,), jnp.float32)

    @pl.loop(0, N_PAD, step=16)
    def _(i):
      hist_v[pl.ds(i, 16)] = zero16

    pltpu.sync_copy(dst_hbm.at[pl.ds(wid * EPT3, EPT3)], dst_v)

    @pl.loop(0, EPT3, step=80)
    def _(t):
      for u in (0, 16, 32, 48, 64):
        idx = dst_v[pl.ds(t + u, 16)]
        cnt, last = plsc.scan_count(idx)
        plsc.addupdate_scatter(hist_v, [idx], cnt.astype(jnp.float32),
                               mask=last)

    pltpu.sync_copy(hist_v, hist_sh.at[s])
    plsc.subcore_barrier()

    @pl.loop(0, RPS, step=16)
    def _(i):
      acc_v[pl.ds(i, 16)] = zero16

    for j in range(NS):
      pltpu.sync_copy(hist_sh.at[j].at[pl.ds(s * RPS, RPS)], tmp_v)

      @pl.loop(0, RPS, step=16)
      def _(i):
        acc_v[pl.ds(i, 16)] += tmp_v[pl.ds(i, 16)]

    pltpu.sync_copy(acc_v, hist_hbm.at[c].at[pl.ds(s * RPS, RPS)])

  return k(dst1d)


def _sc_agg(table, src2d, dst2d, hist2, zeros):
  @functools.partial(
      pl.kernel,
      out_type=jax.ShapeDtypeStruct((NC, N_PAD, D_HID), jnp.float32),
      mesh=_sc_mesh(),
      compiler_params=_SC_PARAMS,
      scratch_types=[
          pltpu.VMEM((NCHUNK, CHUNK), jnp.int32),
          pltpu.VMEM((NCHUNK, CHUNK), jnp.int32),
          pltpu.VMEM((KB, CHUNK, D_HID), jnp.float32),
          pltpu.VMEM((KB, CHUNK, D_HID), jnp.float32),
          pltpu.VMEM((RPS, D_HID), jnp.float32),
          pltpu.VMEM((RPS, D_HID), jnp.float32),
          pltpu.VMEM((RPS,), jnp.float32),
          pltpu.VMEM((RPS,), jnp.float32),
          pltpu.VMEM_SHARED((N_PAD, D_HID), jnp.float32),
          pltpu.VMEM_SHARED((N_PAD, D_HID), jnp.float32),
          pltpu.SemaphoreType.DMA,
          pltpu.SemaphoreType.DMA,
      ],
  )
  def k(tab_hbm, src_hbm, dst_hbm, hist_hbm, z_hbm, out_hbm, src_v, dst_v,
        buf_a, buf_b, tab_v, acc_v, dis_v, tmp_v, acc_sh, tab_sh, sem_a,
        sem_b):
    c = lax.axis_index("c")
    s = lax.axis_index("s")
    wid = c * NS + s
    rows = pl.ds(s * RPS, RPS)

    pltpu.sync_copy(z_hbm.at[rows], acc_sh.at[rows])
    pltpu.sync_copy(tab_hbm.at[rows], tab_v)
    pltpu.sync_copy(hist_hbm.at[0].at[rows], dis_v)
    pltpu.sync_copy(hist_hbm.at[1].at[rows], tmp_v)
    pltpu.sync_copy(src_hbm.at[pl.ds(wid * NCHUNK, NCHUNK)], src_v)
    pltpu.sync_copy(dst_hbm.at[pl.ds(wid * NCHUNK, NCHUNK)], dst_v)

    @pl.loop(0, RPS, step=16)
    def _(i):
      dis_v[pl.ds(i, 16)] = _rsqrt16(dis_v[pl.ds(i, 16)]
                                     + tmp_v[pl.ds(i, 16)] + 1.0)

    @pl.loop(0, RPS, step=2)
    def _(r):
      for u in (0, 1):
        dr = plsc.load_gather(dis_v, [jnp.full((16,), r + u, jnp.int32)])
        tab_v[r + u, :] = tab_v[r + u, :] * dr

    pltpu.sync_copy(tab_v, tab_sh.at[rows])
    plsc.subcore_barrier()

    def fire_g(b, bufs, sem):
      for t in range(KB):
        pltpu.async_copy(tab_sh.at[src_v.at[b * KB + t]], bufs.at[t], sem)

    def drain_g(bufs, sem):
      for t in range(KB):
        pltpu.make_async_copy(tab_hbm.at[src_v.at[0]], bufs.at[t], sem).wait()

    def scatter(b, bufs):
      for t in range(KB):
        pltpu.sync_copy(bufs.at[t], acc_sh.at[dst_v.at[b * KB + t]], add=True)

    fire_g(0, buf_a, sem_a)

    @pl.loop(0, NBLK - 2, step=2)
    def _(b):
      fire_g(b + 1, buf_b, sem_b)
      drain_g(buf_a, sem_a)
      scatter(b, buf_a)
      fire_g(b + 2, buf_a, sem_a)
      drain_g(buf_b, sem_b)
      scatter(b + 1, buf_b)

    fire_g(NBLK - 1, buf_b, sem_b)
    drain_g(buf_a, sem_a)
    scatter(NBLK - 2, buf_a)
    drain_g(buf_b, sem_b)
    scatter(NBLK - 1, buf_b)

    plsc.subcore_barrier()

    pltpu.sync_copy(acc_sh.at[rows], acc_v)
    f16 = jnp.where(jnp.broadcast_to(c, (16,)) == 0, 1.0, 0.0)

    @pl.loop(0, RPS, step=2)
    def _(r):
      for u in (0, 1):
        dr = plsc.load_gather(dis_v, [jnp.full((16,), r + u, jnp.int32)])
        acc_v[r + u, :] = (acc_v[r + u, :] + f16 * tab_v[r + u, :]) * dr

    pltpu.sync_copy(acc_v, out_hbm.at[c].at[rows])

  return k(table, src2d, dst2d, hist2, zeros)


def _tc_mm(xv, w1b):
  def body(x_ref, w_ref, o_ref):
    o_ref[...] = jnp.dot(x_ref[...], w_ref[...],
                         preferred_element_type=jnp.float32)

  return pl.pallas_call(
      body,
      out_shape=jax.ShapeDtypeStruct((VR, 128), jnp.float32),
  )(xv, w1b)


def _tc_mid(q1, b1t):
  def body(q_ref, b_ref, o_ref):
    o_ref[...] = jnp.maximum(q_ref[0] + q_ref[1] + b_ref[...], 0.0)

  return pl.pallas_call(
      body,
      out_shape=jax.ShapeDtypeStruct((VR, 128), jnp.float32),
  )(q1, b1t)


def _tc_final(q2, w2b, b2t):
  def body(q_ref, w_ref, b_ref, o_ref):
    ah = q_ref[0] + q_ref[1]
    o_ref[...] = jnp.dot(ah, w_ref[...],
                         preferred_element_type=jnp.float32) + b_ref[...]

  return pl.pallas_call(
      body,
      out_shape=jax.ShapeDtypeStruct((VR, 8 * D_OUT), jnp.float32),
  )(q2, w2b, b2t)


def _block_diag(w, n):
  return jnp.kron(jnp.eye(n, dtype=w.dtype), w)


@jax.jit
def kernel(x, edge_index, W1, b1, W2, b2):
  x_pad = jnp.pad(x, ((0, N_PAD - N_NODES), (0, 0)))
  xv = x_pad.reshape(VR, 8 * D_IN)
  edges = jnp.pad(edge_index, ((0, 0), (0, E_PAD - N_EDGES)),
                  constant_values=N_PAD - 1)
  src2 = edges[0].reshape(NW * NCHUNK, CHUNK)
  dst2 = edges[1].reshape(NW * NCHUNK, CHUNK)
  z16 = jnp.zeros((N_PAD, D_HID), jnp.float32)
  w1b = _block_diag(W1, 8)
  w2b = _block_diag(W2, 8)
  b1t = jnp.tile(b1, 8).reshape(1, 128)
  b2t = jnp.tile(b2, 8).reshape(1, 8 * D_OUT)

  hist2 = _sc_deg(edge_index[1])
  xw1 = _tc_mm(xv, w1b)
  q1 = _sc_agg(xw1.reshape(N_PAD, D_HID), src2, dst2, hist2, z16)
  h = _tc_mid(q1.reshape(NC, VR, 128), b1t)
  q2 = _sc_agg(h.reshape(N_PAD, D_HID), src2, dst2, hist2, z16)
  out = _tc_final(q2.reshape(NC, VR, 128), w2b, b2t)
  return out.reshape(N_PAD, D_OUT)[:N_NODES]


# speedup vs baseline: 1.0696x; 1.0580x over previous
import functools

import jax
import jax.numpy as jnp
from jax import lax
from jax.experimental import pallas as pl
from jax.experimental.pallas import tpu as pltpu
from jax.experimental.pallas import tpu_sc as plsc

N_NODES = 10000
N_PAD = 10240
D_IN = 128
D_HID = 16
D_OUT = 3
N_EDGES = 320000
NC = 2
NS = 16
NW = NC * NS
CHUNK = 128
NCHUNK = 80
EPT = CHUNK * NCHUNK
E_PAD = NW * EPT
EPT3 = N_EDGES // NW
RPS = N_PAD // NS
VR = N_PAD // 8
KB = 4
NBLK = NCHUNK // KB


def _sc_mesh():
  return plsc.VectorSubcoreMesh(core_axis_name="c", subcore_axis_name="s")


_SC_PARAMS = pltpu.CompilerParams(needs_layout_passes=False,
                                  use_tc_tiling_on_sc=False)


def _rsqrt16(d):
  bits = plsc.bitcast(d, jnp.int32)
  y = plsc.bitcast(jnp.int32(0x5F3759DF) - (bits >> 1), jnp.float32)
  for _ in range(3):
    y = y * (1.5 - 0.5 * d * y * y)
  return y


def _sc_deg(edge_full):
  @functools.partial(
      pl.kernel,
      out_type=jax.ShapeDtypeStruct((NC, N_PAD), jnp.float32),
      mesh=_sc_mesh(),
      compiler_params=_SC_PARAMS,
      scratch_types=[
          pltpu.VMEM((EPT3,), jnp.int32),
          pltpu.VMEM((N_PAD,), jnp.float32),
          pltpu.VMEM((RPS,), jnp.float32),
          pltpu.VMEM((RPS,), jnp.float32),
          pltpu.VMEM_SHARED((NS, N_PAD), jnp.float32),
      ],
  )
  def k(dst_hbm, hist_hbm, dst_v, hist_v, tmp_v, acc_v, hist_sh):
    c = lax.axis_index("c")
    s = lax.axis_index("s")
    wid = c * NS + s
    zero16 = jnp.zeros((16,), jnp.float32)

    @pl.loop(0, N_PAD, step=16)
    def _(i):
      hist_v[pl.ds(i, 16)] = zero16

    pltpu.sync_copy(dst_hbm.at[1].at[pl.ds(wid * EPT3, EPT3)], dst_v)

    @pl.loop(0, EPT3, step=80)
    def _(t):
      for u in (0, 16, 32, 48, 64):
        idx = dst_v[pl.ds(t + u, 16)]
        cnt, last = plsc.scan_count(idx)
        plsc.addupdate_scatter(hist_v, [idx], cnt.astype(jnp.float32),
                               mask=last)

    pltpu.sync_copy(hist_v, hist_sh.at[s])
    plsc.subcore_barrier()

    @pl.loop(0, RPS, step=16)
    def _(i):
      acc_v[pl.ds(i, 16)] = zero16

    for j in range(NS):
      pltpu.sync_copy(hist_sh.at[j].at[pl.ds(s * RPS, RPS)], tmp_v)

      @pl.loop(0, RPS, step=16)
      def _(i):
        acc_v[pl.ds(i, 16)] += tmp_v[pl.ds(i, 16)]

    pltpu.sync_copy(acc_v, hist_hbm.at[c].at[pl.ds(s * RPS, RPS)])

  return k(edge_full)


def _sc_agg(table, src2d, dst2d, hist2, zeros):
  @functools.partial(
      pl.kernel,
      out_type=jax.ShapeDtypeStruct((NC, N_PAD, D_HID), jnp.float32),
      mesh=_sc_mesh(),
      compiler_params=_SC_PARAMS,
      scratch_types=[
          pltpu.VMEM((NCHUNK, CHUNK), jnp.int32),
          pltpu.VMEM((NCHUNK, CHUNK), jnp.int32),
          pltpu.VMEM((KB, CHUNK, D_HID), jnp.float32),
          pltpu.VMEM((KB, CHUNK, D_HID), jnp.float32),
          pltpu.VMEM((RPS, D_HID), jnp.float32),
          pltpu.VMEM((RPS, D_HID), jnp.float32),
          pltpu.VMEM((RPS,), jnp.float32),
          pltpu.VMEM((RPS,), jnp.float32),
          pltpu.VMEM_SHARED((N_PAD, D_HID), jnp.float32),
          pltpu.VMEM_SHARED((N_PAD, D_HID), jnp.float32),
          pltpu.SemaphoreType.DMA,
          pltpu.SemaphoreType.DMA,
      ],
  )
  def k(tab_hbm, src_hbm, dst_hbm, hist_hbm, z_hbm, out_hbm, src_v, dst_v,
        buf_a, buf_b, tab_v, acc_v, dis_v, tmp_v, acc_sh, tab_sh, sem_a,
        sem_b):
    c = lax.axis_index("c")
    s = lax.axis_index("s")
    wid = c * NS + s
    rows = pl.ds(s * RPS, RPS)

    pltpu.sync_copy(z_hbm.at[rows], acc_sh.at[rows])
    pltpu.sync_copy(tab_hbm.at[rows], tab_v)
    pltpu.sync_copy(hist_hbm.at[0].at[rows], dis_v)
    pltpu.sync_copy(hist_hbm.at[1].at[rows], tmp_v)
    pltpu.sync_copy(src_hbm.at[pl.ds(wid * NCHUNK, NCHUNK)], src_v)
    pltpu.sync_copy(dst_hbm.at[pl.ds(wid * NCHUNK, NCHUNK)], dst_v)

    @pl.loop(0, RPS, step=16)
    def _(i):
      dis_v[pl.ds(i, 16)] = _rsqrt16(dis_v[pl.ds(i, 16)]
                                     + tmp_v[pl.ds(i, 16)] + 1.0)

    @pl.loop(0, RPS, step=2)
    def _(r):
      for u in (0, 1):
        dr = plsc.load_gather(dis_v, [jnp.full((16,), r + u, jnp.int32)])
        tab_v[r + u, :] = tab_v[r + u, :] * dr

    pltpu.sync_copy(tab_v, tab_sh.at[rows])
    plsc.subcore_barrier()

    def fire_g(b, bufs, sem):
      for t in range(KB):
        pltpu.async_copy(tab_sh.at[src_v.at[b * KB + t]], bufs.at[t], sem)

    def drain_g(bufs, sem):
      for t in range(KB):
        pltpu.make_async_copy(tab_hbm.at[src_v.at[0]], bufs.at[t], sem).wait()

    def scatter(b, bufs):
      for t in range(KB):
        pltpu.sync_copy(bufs.at[t], acc_sh.at[dst_v.at[b * KB + t]], add=True)

    fire_g(0, buf_a, sem_a)

    @pl.loop(0, NBLK - 2, step=2)
    def _(b):
      fire_g(b + 1, buf_b, sem_b)
      drain_g(buf_a, sem_a)
      scatter(b, buf_a)
      fire_g(b + 2, buf_a, sem_a)
      drain_g(buf_b, sem_b)
      scatter(b + 1, buf_b)

    fire_g(NBLK - 1, buf_b, sem_b)
    drain_g(buf_a, sem_a)
    scatter(NBLK - 2, buf_a)
    drain_g(buf_b, sem_b)
    scatter(NBLK - 1, buf_b)

    plsc.subcore_barrier()

    pltpu.sync_copy(acc_sh.at[rows], acc_v)
    f16 = jnp.where(jnp.broadcast_to(c, (16,)) == 0, 1.0, 0.0)

    @pl.loop(0, RPS, step=2)
    def _(r):
      for u in (0, 1):
        dr = plsc.load_gather(dis_v, [jnp.full((16,), r + u, jnp.int32)])
        acc_v[r + u, :] = (acc_v[r + u, :] + f16 * tab_v[r + u, :]) * dr

    pltpu.sync_copy(acc_v, out_hbm.at[c].at[rows])

  return k(table, src2d, dst2d, hist2, zeros)


def _tc_mm(xv, w1):
  def body(x_ref, w_ref, o_ref):
    w = w_ref[...]
    for r in range(8):
      o_ref[:, 16 * r:16 * (r + 1)] = jnp.dot(
          x_ref[:, 128 * r:128 * (r + 1)], w,
          preferred_element_type=jnp.float32)

  return pl.pallas_call(
      body,
      out_shape=jax.ShapeDtypeStruct((VR, 128), jnp.float32),
  )(xv, w1)


def _tc_mid(q1, b1t):
  def body(q_ref, b_ref, o_ref):
    o_ref[...] = jnp.maximum(q_ref[0] + q_ref[1] + b_ref[...], 0.0)

  return pl.pallas_call(
      body,
      out_shape=jax.ShapeDtypeStruct((VR, 128), jnp.float32),
  )(q1, b1t)


def _tc_final(q2, w2, b2r):
  def body(q_ref, w_ref, b_ref, o_ref):
    ah = q_ref[0] + q_ref[1]
    w = w_ref[...]
    b = b_ref[...]
    for r in range(8):
      o_ref[:, D_OUT * r:D_OUT * (r + 1)] = jnp.dot(
          ah[:, 16 * r:16 * (r + 1)], w,
          preferred_element_type=jnp.float32) + b

  return pl.pallas_call(
      body,
      out_shape=jax.ShapeDtypeStruct((VR, 8 * D_OUT), jnp.float32),
  )(q2, w2, b2r)


@jax.jit
def kernel(x, edge_index, W1, b1, W2, b2):
  x_pad = jnp.pad(x, ((0, N_PAD - N_NODES), (0, 0)))
  xv = x_pad.reshape(VR, 8 * D_IN)
  edges = jnp.pad(edge_index, ((0, 0), (0, E_PAD - N_EDGES)),
                  constant_values=N_PAD - 1)
  src2 = edges[0].reshape(NW * NCHUNK, CHUNK)
  dst2 = edges[1].reshape(NW * NCHUNK, CHUNK)
  z16 = jnp.zeros((N_PAD, D_HID), jnp.float32)
  b1t = jnp.tile(b1, 8).reshape(1, 128)

  hist2 = _sc_deg(edge_index)
  xw1 = _tc_mm(xv, W1)
  q1 = _sc_agg(xw1.reshape(N_PAD, D_HID), src2, dst2, hist2, z16)
  h = _tc_mid(q1.reshape(NC, VR, 128), b1t)
  q2 = _sc_agg(h.reshape(N_PAD, D_HID), src2, dst2, hist2, z16)
  out = _tc_final(q2.reshape(NC, VR, 128), W2, b2.reshape(1, D_OUT))
  return out.reshape(N_PAD, D_OUT)[:N_NODES]
